# baseline (speedup 1.0000x reference)
import jax
import jax.numpy as jnp
from jax.experimental import pallas as pl
from jax.experimental.pallas import tpu as pltpu

_NEG = float(-jnp.finfo(jnp.bfloat16).max)

_BT = 256
_CB = 256
_E = 8


def _router_body(logits_ref, gates_ref, pre_ref, mask_ref, rw_ref):
    logits = logits_ref[...]
    gates = gates_ref[...]
    e = logits.shape[1]

    pre = logits - jax.nn.sigmoid(gates)
    gated = jnp.maximum(pre, 0.0)
    amask = (gated > 0.0).astype(jnp.float32)
    num_active = jnp.sum(amask, axis=1, keepdims=True)

    vk = logits[:, None, :]
    vj = logits[:, :, None]
    kk = jax.lax.broadcasted_iota(jnp.int32, (1, e, e), 2)
    jj = jax.lax.broadcasted_iota(jnp.int32, (1, e, e), 1)
    beats = (vk > vj) | ((vk == vj) & (kk < jj))
    rank = jnp.sum(beats.astype(jnp.float32), axis=2)
    fb = (rank < (e // 2)).astype(jnp.float32)

    mask = jnp.where(num_active == 0.0, fb, amask)
    gm = jnp.where(mask > 0.0, gated, _NEG)
    gmax = jnp.max(gm, axis=1, keepdims=True)
    ex = jnp.exp(gm - gmax)
    rw = ex / jnp.sum(ex, axis=1, keepdims=True)

    pre_ref[...] = pre
    mask_ref[...] = mask
    rw_ref[...] = rw


def _expert_body(x_ref, w1_ref, w2_ref, mask_ref, rw_ref,
                 fuo_ref, fin_ref, stack_ref):
    s = pl.program_id(1)
    nc = stack_ref.shape[0]
    cb = stack_ref.shape[3]

    @pl.when(s < _E)
    def _compute():
        xb = x_ref[...].astype(jnp.bfloat16)
        w1 = w1_ref[s].astype(jnp.bfloat16)
        w2 = w2_ref[s].astype(jnp.bfloat16)
        h = jax.lax.dot_general(
            xb, w1, (((1,), (1,)), ((), ())),
            preferred_element_type=jnp.float32)
        a = 0.5 * h * (1.0 + jax.lax.erf(h * 0.7071067811865476))
        o = jax.lax.dot_general(
            a.astype(jnp.bfloat16), w2, (((1,), (1,)), ((), ())),
            preferred_element_type=jnp.float32)

        onehot = (jax.lax.broadcasted_iota(jnp.int32, (1, _E), 1) == s
                  ).astype(jnp.float32)
        m = jnp.sum(mask_ref[...] * onehot, axis=1, keepdims=True)
        r = jnp.sum(rw_ref[...] * onehot, axis=1, keepdims=True)

        fuo = m * o
        for c0 in range(nc):
            stack_ref[c0, s] = fuo[:, c0 * cb:(c0 + 1) * cb]
        contrib = r * fuo

        @pl.when(s == 0)
        def _init():
            fin_ref[...] = contrib

        @pl.when(s > 0)
        def _acc():
            fin_ref[...] += contrib

    @pl.when(s >= _E)
    def _flush():
        chunk = stack_ref[s - _E]
        fuo_ref[...] = jnp.swapaxes(chunk, 0, 1)


def kernel(hidden_states, sim_matrix, gates, W1, W2):
    x = hidden_states
    t, c = x.shape
    e = sim_matrix.shape[1]
    i = W1.shape[1]

    xnorm = jnp.linalg.norm(x, axis=-1, keepdims=True)
    snorm = jnp.linalg.norm(sim_matrix, axis=0, keepdims=True)
    logits = (x / jnp.maximum(xnorm, 1e-12)) @ (sim_matrix / jnp.maximum(snorm, 1e-12))

    bt_r = 512
    pre, mask, rw = pl.pallas_call(
        _router_body,
        grid=(t // bt_r,),
        in_specs=[
            pl.BlockSpec((bt_r, e), lambda ti: (ti, 0)),
            pl.BlockSpec((1, e), lambda ti: (0, 0)),
        ],
        out_specs=[
            pl.BlockSpec((bt_r, e), lambda ti: (ti, 0)),
            pl.BlockSpec((bt_r, e), lambda ti: (ti, 0)),
            pl.BlockSpec((bt_r, e), lambda ti: (ti, 0)),
        ],
        out_shape=[
            jax.ShapeDtypeStruct((t, e), jnp.float32),
            jax.ShapeDtypeStruct((t, e), jnp.float32),
            jax.ShapeDtypeStruct((t, e), jnp.float32),
        ],
    )(logits, gates.reshape(1, e))

    nc = c // _CB
    fuo, fin = pl.pallas_call(
        _expert_body,
        grid=(t // _BT, _E + nc),
        in_specs=[
            pl.BlockSpec((_BT, c), lambda ti, s: (ti, 0)),
            pl.BlockSpec((_E, i, c), lambda ti, s: (0, 0, 0)),
            pl.BlockSpec((_E, c, i), lambda ti, s: (0, 0, 0)),
            pl.BlockSpec((_BT, e), lambda ti, s: (ti, 0)),
            pl.BlockSpec((_BT, e), lambda ti, s: (ti, 0)),
        ],
        out_specs=[
            pl.BlockSpec((_BT, e, _CB),
                         lambda ti, s: (ti, 0, jnp.maximum(s - _E, 0))),
            pl.BlockSpec((_BT, c), lambda ti, s: (ti, 0)),
        ],
        out_shape=[
            jax.ShapeDtypeStruct((t, e, c), jnp.float32),
            jax.ShapeDtypeStruct((t, c), jnp.float32),
        ],
        scratch_shapes=[pltpu.VMEM((nc, _E, _BT, _CB), jnp.float32)],
        compiler_params=pltpu.CompilerParams(
            dimension_semantics=("arbitrary", "arbitrary")),
    )(x, W1, W2, mask, rw)

    return (fin, fuo, pre, mask)


# speedup vs baseline: 1.2020x; 1.2020x over previous
import jax
import jax.numpy as jnp
from jax.experimental import pallas as pl
from jax.experimental.pallas import tpu as pltpu

_NEG = float(-jnp.finfo(jnp.bfloat16).max)

_BT = 512
_CB = 256
_E = 8


def _router_body(logits_ref, gates_ref, pre_ref, mask_ref, rw_ref):
    logits = logits_ref[...]
    gates = gates_ref[...]
    e = logits.shape[1]

    pre = logits - jax.nn.sigmoid(gates)
    gated = jnp.maximum(pre, 0.0)
    amask = (gated > 0.0).astype(jnp.float32)
    num_active = jnp.sum(amask, axis=1, keepdims=True)

    jdx = jax.lax.broadcasted_iota(jnp.int32, (1, e), 1)
    rank = jnp.zeros_like(logits)
    for k in range(e):
        vk = logits[:, k:k + 1]
        beats = (vk > logits) | ((vk == logits) & (jdx > k))
        rank = rank + beats.astype(jnp.float32)
    fb = (rank < (e // 2)).astype(jnp.float32)

    mask = jnp.where(num_active == 0.0, fb, amask)
    gm = jnp.where(mask > 0.0, gated, _NEG)
    gmax = jnp.max(gm, axis=1, keepdims=True)
    ex = jnp.exp(gm - gmax)
    rw = ex / jnp.sum(ex, axis=1, keepdims=True)

    pre_ref[...] = pre
    mask_ref[...] = mask
    rw_ref[...] = rw


def _expert_body(x_ref, w1_ref, w2_ref, mask_ref, rw_ref,
                 fuo_ref, fin_ref, stack_ref, w1b_ref, w2b_ref, xb_ref,
                 oprev_ref):
    ti = pl.program_id(0)
    s = pl.program_id(1)
    nc = stack_ref.shape[0]
    cb = stack_ref.shape[3]
    i = w1b_ref.shape[1]

    @pl.when((s < _E) & (ti == 0))
    def _cast_weights():
        w1b_ref[s] = w1_ref[0].astype(jnp.bfloat16)
        w2b_ref[s] = w2_ref[0].astype(jnp.bfloat16)

    @pl.when(s == 0)
    def _cast_x():
        xb_ref[...] = x_ref[...].astype(jnp.bfloat16)

    @pl.when((s > 0) & (s <= _E))
    def _epilogue():
        e_prev = s - 1
        op = oprev_ref[...]
        onehot = (jax.lax.broadcasted_iota(jnp.int32, (1, _E), 1) == e_prev
                  ).astype(jnp.float32)
        m = jnp.sum(mask_ref[...] * onehot, axis=1, keepdims=True)
        r = jnp.sum(rw_ref[...] * onehot, axis=1, keepdims=True)
        fuo = m * op
        for c0 in range(nc):
            stack_ref[c0, e_prev] = fuo[:, c0 * cb:(c0 + 1) * cb].astype(jnp.bfloat16)
        contrib = r * fuo

        @pl.when(s == 1)
        def _init():
            fin_ref[...] = contrib

        @pl.when(s > 1)
        def _acc():
            fin_ref[...] += contrib

    @pl.when(s < _E)
    def _compute():
        xb = xb_ref[...]
        h = jax.lax.dot_general(
            xb, w1b_ref[s], (((1,), (1,)), ((), ())),
            preferred_element_type=jnp.float32)
        hh = i // 2
        o = None
        for k0 in range(2):
            hk = h[:, k0 * hh:(k0 + 1) * hh]
            ak = 0.5 * hk * (1.0 + jax.lax.erf(hk * 0.7071067811865476))
            ok = jax.lax.dot_general(
                ak.astype(jnp.bfloat16),
                w2b_ref[s][:, k0 * hh:(k0 + 1) * hh],
                (((1,), (1,)), ((), ())),
                preferred_element_type=jnp.float32)
            o = ok if o is None else o + ok
        oprev_ref[...] = o

    @pl.when(s >= _E)
    def _flush():
        chunk = stack_ref[s - _E]
        fuo_ref[...] = jnp.swapaxes(chunk, 0, 1).astype(jnp.float32)


def kernel(hidden_states, sim_matrix, gates, W1, W2):
    x = hidden_states
    t, c = x.shape
    e = sim_matrix.shape[1]
    i = W1.shape[1]

    xnorm = jnp.linalg.norm(x, axis=-1, keepdims=True)
    snorm = jnp.linalg.norm(sim_matrix, axis=0, keepdims=True)
    logits = (x / jnp.maximum(xnorm, 1e-12)) @ (sim_matrix / jnp.maximum(snorm, 1e-12))

    bt_r = t
    pre, mask, rw = pl.pallas_call(
        _router_body,
        grid=(t // bt_r,),
        in_specs=[
            pl.BlockSpec((bt_r, e), lambda ti: (ti, 0)),
            pl.BlockSpec((1, e), lambda ti: (0, 0)),
        ],
        out_specs=[
            pl.BlockSpec((bt_r, e), lambda ti: (ti, 0)),
            pl.BlockSpec((bt_r, e), lambda ti: (ti, 0)),
            pl.BlockSpec((bt_r, e), lambda ti: (ti, 0)),
        ],
        out_shape=[
            jax.ShapeDtypeStruct((t, e), jnp.float32),
            jax.ShapeDtypeStruct((t, e), jnp.float32),
            jax.ShapeDtypeStruct((t, e), jnp.float32),
        ],
    )(logits, gates.reshape(1, e))

    nc = c // _CB
    fuo, fin = pl.pallas_call(
        _expert_body,
        grid=(t // _BT, _E + nc),
        in_specs=[
            pl.BlockSpec((_BT, c), lambda ti, s: (ti, 0)),
            pl.BlockSpec((1, i, c),
                         lambda ti, s: (jnp.where(ti == 0, jnp.minimum(s, _E - 1),
                                                  _E - 1), 0, 0)),
            pl.BlockSpec((1, c, i),
                         lambda ti, s: (jnp.where(ti == 0, jnp.minimum(s, _E - 1),
                                                  _E - 1), 0, 0)),
            pl.BlockSpec((_BT, e), lambda ti, s: (ti, 0)),
            pl.BlockSpec((_BT, e), lambda ti, s: (ti, 0)),
        ],
        out_specs=[
            pl.BlockSpec((_BT, e, _CB),
                         lambda ti, s: (ti, 0, jnp.maximum(s - _E, 0))),
            pl.BlockSpec((_BT, c), lambda ti, s: (ti, 0)),
        ],
        out_shape=[
            jax.ShapeDtypeStruct((t, e, c), jnp.float32),
            jax.ShapeDtypeStruct((t, c), jnp.float32),
        ],
        scratch_shapes=[
            pltpu.VMEM((nc, _E, _BT, _CB), jnp.bfloat16),
            pltpu.VMEM((_E, i, c), jnp.bfloat16),
            pltpu.VMEM((_E, c, i), jnp.bfloat16),
            pltpu.VMEM((_BT, c), jnp.bfloat16),
            pltpu.VMEM((_BT, c), jnp.float32),
        ],
        compiler_params=pltpu.CompilerParams(
            dimension_semantics=("arbitrary", "arbitrary")),
    )(x, W1, W2, mask, rw)

    return (fin, fuo, pre, mask)


# speedup vs baseline: 1.3254x; 1.1027x over previous
import jax
import jax.numpy as jnp
from jax.experimental import pallas as pl
from jax.experimental.pallas import tpu as pltpu

_NEG = float(-jnp.finfo(jnp.bfloat16).max)

_BT = 512
_CB = 256
_E = 8


def _router_body(logits_ref, gates_ref, pre_ref, mask_ref, rw_ref):
    logits = logits_ref[...]
    gates = gates_ref[...]
    e = logits.shape[1]

    pre = logits - jax.nn.sigmoid(gates)
    gated = jnp.maximum(pre, 0.0)
    amask = (gated > 0.0).astype(jnp.float32)
    num_active = jnp.sum(amask, axis=1, keepdims=True)

    jdx = jax.lax.broadcasted_iota(jnp.int32, (1, e), 1)
    rank = jnp.zeros_like(logits)
    for k in range(e):
        vk = logits[:, k:k + 1]
        beats = (vk > logits) | ((vk == logits) & (jdx > k))
        rank = rank + beats.astype(jnp.float32)
    fb = (rank < (e // 2)).astype(jnp.float32)

    mask = jnp.where(num_active == 0.0, fb, amask)
    gm = jnp.where(mask > 0.0, gated, _NEG)
    gmax = jnp.max(gm, axis=1, keepdims=True)
    ex = jnp.exp(gm - gmax)
    rw = ex / jnp.sum(ex, axis=1, keepdims=True)

    pre_ref[...] = pre
    mask_ref[...] = mask
    rw_ref[...] = rw


def _expert_body(x_ref, w1_ref, w2_ref, mask_ref, rw_ref,
                 fuo_ref, fin_ref, stack_ref, w1b_ref, w2b_ref, xb_ref,
                 oprev_ref):
    ti = pl.program_id(0)
    s = pl.program_id(1)
    nc = stack_ref.shape[0]
    cb = stack_ref.shape[3]
    i = w1b_ref.shape[1]

    @pl.when((s < _E) & (ti == 0))
    def _cast_weights():
        w1b_ref[s] = w1_ref[0].astype(jnp.bfloat16)
        w2b_ref[s] = w2_ref[0].astype(jnp.bfloat16)

    @pl.when(s == 0)
    def _cast_x():
        xb_ref[...] = x_ref[...].astype(jnp.bfloat16)

    def _gemms(e_idx):
        xb = xb_ref[...]
        h = jax.lax.dot_general(
            xb, w1b_ref[e_idx], (((1,), (1,)), ((), ())),
            preferred_element_type=jnp.float32)
        a = 0.5 * h * (1.0 + jax.lax.erf(h * 0.7071067811865476))
        o = jax.lax.dot_general(
            a.astype(jnp.bfloat16), w2b_ref[e_idx], (((1,), (1,)), ((), ())),
            preferred_element_type=jnp.float32)
        oprev_ref[...] = o

    def _epilogue(e_prev):
        op = oprev_ref[...]
        onehot = (jax.lax.broadcasted_iota(jnp.int32, (1, _E), 1) == e_prev
                  ).astype(jnp.float32)
        m = jnp.sum(mask_ref[...] * onehot, axis=1, keepdims=True)
        r = jnp.sum(rw_ref[...] * onehot, axis=1, keepdims=True)
        fuo = m * op
        for c0 in range(nc):
            stack_ref[c0, e_prev] = fuo[:, c0 * cb:(c0 + 1) * cb].astype(jnp.bfloat16)
        fin_ref[...] += r * fuo

    @pl.when(s == 0)
    def _first():
        fin_ref[...] = jnp.zeros(fin_ref.shape, fin_ref.dtype)
        _gemms(s)

    @pl.when((s > 0) & (s < _E))
    def _steady():
        _epilogue(s - 1)
        _gemms(s)

    @pl.when(s >= _E)
    def _flush():
        @pl.when(s == _E)
        def _last_epilogue():
            _epilogue(_E - 1)

        chunk = stack_ref[s - _E]
        fuo_ref[...] = jnp.swapaxes(chunk, 0, 1).astype(jnp.float32)


def kernel(hidden_states, sim_matrix, gates, W1, W2):
    x = hidden_states
    t, c = x.shape
    e = sim_matrix.shape[1]
    i = W1.shape[1]

    xnorm = jnp.linalg.norm(x, axis=-1, keepdims=True)
    snorm = jnp.linalg.norm(sim_matrix, axis=0, keepdims=True)
    logits = (x / jnp.maximum(xnorm, 1e-12)) @ (sim_matrix / jnp.maximum(snorm, 1e-12))

    bt_r = t
    pre, mask, rw = pl.pallas_call(
        _router_body,
        grid=(t // bt_r,),
        in_specs=[
            pl.BlockSpec((bt_r, e), lambda ti: (ti, 0)),
            pl.BlockSpec((1, e), lambda ti: (0, 0)),
        ],
        out_specs=[
            pl.BlockSpec((bt_r, e), lambda ti: (ti, 0)),
            pl.BlockSpec((bt_r, e), lambda ti: (ti, 0)),
            pl.BlockSpec((bt_r, e), lambda ti: (ti, 0)),
        ],
        out_shape=[
            jax.ShapeDtypeStruct((t, e), jnp.float32),
            jax.ShapeDtypeStruct((t, e), jnp.float32),
            jax.ShapeDtypeStruct((t, e), jnp.float32),
        ],
    )(logits, gates.reshape(1, e))

    nc = c // _CB
    fuo, fin = pl.pallas_call(
        _expert_body,
        grid=(t // _BT, _E + nc),
        in_specs=[
            pl.BlockSpec((_BT, c), lambda ti, s: (ti, 0)),
            pl.BlockSpec((1, i, c),
                         lambda ti, s: (jnp.where(ti == 0, jnp.minimum(s, _E - 1),
                                                  _E - 1), 0, 0)),
            pl.BlockSpec((1, c, i),
                         lambda ti, s: (jnp.where(ti == 0, jnp.minimum(s, _E - 1),
                                                  _E - 1), 0, 0)),
            pl.BlockSpec((_BT, e), lambda ti, s: (ti, 0)),
            pl.BlockSpec((_BT, e), lambda ti, s: (ti, 0)),
        ],
        out_specs=[
            pl.BlockSpec((_BT, e, _CB),
                         lambda ti, s: (ti, 0, jnp.maximum(s - _E, 0))),
            pl.BlockSpec((_BT, c), lambda ti, s: (ti, 0)),
        ],
        out_shape=[
            jax.ShapeDtypeStruct((t, e, c), jnp.float32),
            jax.ShapeDtypeStruct((t, c), jnp.float32),
        ],
        scratch_shapes=[
            pltpu.VMEM((nc, _E, _BT, _CB), jnp.bfloat16),
            pltpu.VMEM((_E, i, c), jnp.bfloat16),
            pltpu.VMEM((_E, c, i), jnp.bfloat16),
            pltpu.VMEM((_BT, c), jnp.bfloat16),
            pltpu.VMEM((_BT, c), jnp.float32),
        ],
        compiler_params=pltpu.CompilerParams(
            dimension_semantics=("arbitrary", "arbitrary")),
    )(x, W1, W2, mask, rw)

    return (fin, fuo, pre, mask)


# speedup vs baseline: 1.3841x; 1.0443x over previous
import jax
import jax.numpy as jnp
from jax.experimental import pallas as pl
from jax.experimental.pallas import tpu as pltpu

_NEG = float(-jnp.finfo(jnp.bfloat16).max)

_BT = 512
_CB = 256
_E = 8


def _expert_body(x_ref, w1_ref, w2_ref, lg_ref, gates_ref,
                 fuo_ref, fin_ref, pre_ref, mask_ref,
                 stack_ref, w1b_ref, w2b_ref, xb_ref, oprev_ref, rw_ref):
    ti = pl.program_id(0)
    s = pl.program_id(1)
    nc = stack_ref.shape[0]
    cb = stack_ref.shape[3]

    @pl.when((s < _E) & (ti == 0))
    def _cast_weights():
        w1b_ref[s] = w1_ref[0].astype(jnp.bfloat16)
        w2b_ref[s] = w2_ref[0].astype(jnp.bfloat16)

    def _router():
        logits = lg_ref[...]
        gates = gates_ref[...]

        pre = logits - jax.nn.sigmoid(gates)
        gated = jnp.maximum(pre, 0.0)
        amask = (gated > 0.0).astype(jnp.float32)
        num_active = jnp.sum(amask, axis=1, keepdims=True)

        jdx = jax.lax.broadcasted_iota(jnp.int32, (1, _E), 1)
        rank = jnp.zeros_like(logits)
        for k in range(_E):
            vk = logits[:, k:k + 1]
            beats = (vk > logits) | ((vk == logits) & (jdx > k))
            rank = rank + beats.astype(jnp.float32)
        fb = (rank < (_E // 2)).astype(jnp.float32)

        mask = jnp.where(num_active == 0.0, fb, amask)
        gm = jnp.where(mask > 0.0, gated, _NEG)
        gmax = jnp.max(gm, axis=1, keepdims=True)
        ex = jnp.exp(gm - gmax)
        rw = ex / jnp.sum(ex, axis=1, keepdims=True)

        pre_ref[...] = pre
        mask_ref[...] = mask
        rw_ref[...] = rw

    def _gemms(e_idx):
        xb = xb_ref[...]
        h = jax.lax.dot_general(
            xb, w1b_ref[e_idx], (((1,), (1,)), ((), ())),
            preferred_element_type=jnp.float32)
        a = 0.5 * h * (1.0 + jax.lax.erf(h * 0.7071067811865476))
        o = jax.lax.dot_general(
            a.astype(jnp.bfloat16), w2b_ref[e_idx], (((1,), (1,)), ((), ())),
            preferred_element_type=jnp.float32)
        oprev_ref[...] = o

    def _epilogue(e_prev):
        op = oprev_ref[...]
        onehot = (jax.lax.broadcasted_iota(jnp.int32, (1, _E), 1) == e_prev
                  ).astype(jnp.float32)
        m = jnp.sum(mask_ref[...] * onehot, axis=1, keepdims=True)
        r = jnp.sum(rw_ref[...] * onehot, axis=1, keepdims=True)
        fuo = m * op
        for c0 in range(nc):
            stack_ref[c0, e_prev] = fuo[:, c0 * cb:(c0 + 1) * cb].astype(jnp.bfloat16)
        fin_ref[...] += r * fuo

    @pl.when(s == 0)
    def _first():
        _router()
        xb_ref[...] = x_ref[...].astype(jnp.bfloat16)
        fin_ref[...] = jnp.zeros(fin_ref.shape, fin_ref.dtype)
        _gemms(s)

    @pl.when((s > 0) & (s < _E))
    def _steady():
        _epilogue(s - 1)
        _gemms(s)

    @pl.when(s >= _E)
    def _flush():
        @pl.when(s == _E)
        def _last_epilogue():
            _epilogue(_E - 1)

        chunk = stack_ref[s - _E]
        fuo_ref[...] = jnp.swapaxes(chunk, 0, 1).astype(jnp.float32)


def kernel(hidden_states, sim_matrix, gates, W1, W2):
    x = hidden_states
    t, c = x.shape
    e = sim_matrix.shape[1]
    i = W1.shape[1]

    xnorm = jnp.linalg.norm(x, axis=-1, keepdims=True)
    snorm = jnp.linalg.norm(sim_matrix, axis=0, keepdims=True)
    logits = (x / jnp.maximum(xnorm, 1e-12)) @ (sim_matrix / jnp.maximum(snorm, 1e-12))

    nc = c // _CB
    fuo, fin, pre, mask = pl.pallas_call(
        _expert_body,
        grid=(t // _BT, _E + nc),
        in_specs=[
            pl.BlockSpec((_BT, c), lambda ti, s: (ti, 0)),
            pl.BlockSpec((1, i, c),
                         lambda ti, s: (jnp.where(ti == 0, jnp.minimum(s, _E - 1),
                                                  _E - 1), 0, 0)),
            pl.BlockSpec((1, c, i),
                         lambda ti, s: (jnp.where(ti == 0, jnp.minimum(s, _E - 1),
                                                  _E - 1), 0, 0)),
            pl.BlockSpec((_BT, e), lambda ti, s: (ti, 0)),
            pl.BlockSpec((1, e), lambda ti, s: (0, 0)),
        ],
        out_specs=[
            pl.BlockSpec((_BT, e, _CB),
                         lambda ti, s: (ti, 0, jnp.maximum(s - _E, 0))),
            pl.BlockSpec((_BT, c), lambda ti, s: (ti, 0)),
            pl.BlockSpec((_BT, e), lambda ti, s: (ti, 0)),
            pl.BlockSpec((_BT, e), lambda ti, s: (ti, 0)),
        ],
        out_shape=[
            jax.ShapeDtypeStruct((t, e, c), jnp.float32),
            jax.ShapeDtypeStruct((t, c), jnp.float32),
            jax.ShapeDtypeStruct((t, e), jnp.float32),
            jax.ShapeDtypeStruct((t, e), jnp.float32),
        ],
        scratch_shapes=[
            pltpu.VMEM((nc, _E, _BT, _CB), jnp.bfloat16),
            pltpu.VMEM((_E, i, c), jnp.bfloat16),
            pltpu.VMEM((_E, c, i), jnp.bfloat16),
            pltpu.VMEM((_BT, c), jnp.bfloat16),
            pltpu.VMEM((_BT, c), jnp.float32),
            pltpu.VMEM((_BT, _E), jnp.float32),
        ],
        compiler_params=pltpu.CompilerParams(
            dimension_semantics=("arbitrary", "arbitrary")),
    )(x, W1, W2, logits, gates.reshape(1, e))

    return (fin, fuo, pre, mask)
